# baseline (speedup 1.0000x reference)
import functools

import jax
import jax.numpy as jnp
from jax import lax
from jax.experimental import pallas as pl
from jax.experimental.pallas import tpu as pltpu
from jax.experimental.pallas import tpu_sc as plsc

_B = 128
_NBUF = 4
_AW = 80


def _mm_body(F, x_ref, w_ref, b_ref, ei_ref, xw1_ref, z_ref,
             src_ref, dst_ref):
    x = x_ref[...]
    w = w_ref[...]
    n = x.shape[0]
    xw1_ref[...] = (
        jnp.dot(x, w[:F, :], preferred_element_type=jnp.float32) + b_ref[...]
    )
    z = jnp.dot(x, w[F:, :], preferred_element_type=jnp.float32)
    ch = z.shape[1]
    pad = jnp.zeros((n, _AW - ch - 1), jnp.float32)
    ones = jnp.ones((n, 1), jnp.float32)
    z_ref[...] = jnp.concatenate([z, ones, pad], axis=-1)
    i = pl.program_id(0)
    e = ei_ref[...]
    eblk = e.shape[1]
    src_ref[pl.ds(i * eblk, eblk)] = e[0]
    dst_ref[pl.ds(i * eblk, eblk)] = e[1]


def _phase1(x, W, b2, ei):
    N, F = x.shape
    CH = W.shape[1]
    E = ei.shape[1]
    g = 5
    blk = N // g
    eblk = E // g
    assert N % g == 0 and E % g == 0 and blk % 8 == 0
    return pl.pallas_call(
        functools.partial(_mm_body, F),
        grid=(g,),
        in_specs=[
            pl.BlockSpec((blk, F), lambda i: (i, 0)),
            pl.BlockSpec((2 * F, CH), lambda i: (0, 0)),
            pl.BlockSpec((1, CH), lambda i: (0, 0)),
            pl.BlockSpec((2, eblk), lambda i: (0, i)),
        ],
        out_specs=(
            pl.BlockSpec((blk, CH), lambda i: (i, 0)),
            pl.BlockSpec((blk, _AW), lambda i: (i, 0)),
            pl.BlockSpec((E,), lambda i: (0,)),
            pl.BlockSpec((E,), lambda i: (0,)),
        ),
        out_shape=(
            jax.ShapeDtypeStruct((N, CH), jnp.float32),
            jax.ShapeDtypeStruct((N, _AW), jnp.float32),
            jax.ShapeDtypeStruct((E,), jnp.int32),
            jax.ShapeDtypeStruct((E,), jnp.int32),
        ),
    )(x, W, b2, ei)


def _sc_body(
    N, NC, NS, ept,
    z_hbm, src_hbm, dst_hbm,
    seg_out,
    srcv, dstv, dring, dtail, rows, rtail, zbuf,
    acc, gsems, ssems, psem,
):
    cid = lax.axis_index("c")
    sid = lax.axis_index("s")
    wid = sid * NC + cid

    n_full = ept // _B
    tail = ept - n_full * _B

    zvec = jnp.zeros((16,), jnp.float32)

    ebase = pl.multiple_of(wid * ept, 8)
    pltpu.async_copy(src_hbm.at[pl.ds(ebase, ept)], srcv, psem)
    pltpu.async_copy(dst_hbm.at[pl.ds(ebase, ept)], dstv, psem)

    for r in range(128):
        for k in range(_AW // 16):
            zbuf[r, pl.ds(k * 16, 16)] = zvec

    n_zfull = N // 128
    z_tail = N - n_zfull * 128
    n_zchunk = n_zfull + (1 if z_tail else 0)
    z_iters = (n_zchunk + NS - 1) // NS

    def _zrun(start):
        def _go(i, _):
            ck = i * NS + sid

            @pl.when(ck < n_zfull)
            def _():
                d = pltpu.make_async_copy(
                    zbuf, acc.at[pl.ds(ck * 128, 128)], ssems.at[0])
                d.start() if start else d.wait()

            if z_tail:
                @pl.when(ck == n_zfull)
                def _():
                    d = pltpu.make_async_copy(
                        zbuf.at[pl.ds(0, z_tail)],
                        acc.at[pl.ds(n_zfull * 128, z_tail)], ssems.at[0])
                    d.start() if start else d.wait()

            return 0

        lax.fori_loop(0, z_iters, _go, 0)

    _zrun(True)
    _zrun(False)

    pltpu.make_async_copy(src_hbm.at[pl.ds(ebase, ept)], srcv, psem).wait()
    pltpu.make_async_copy(dst_hbm.at[pl.ds(ebase, ept)], dstv, psem).wait()

    plsc.subcore_barrier()

    def _stage_dst(c, b):
        for h in range(2):
            for k in range(_B // 32):
                dring[b, h, pl.ds(k * 16, 16)] = dstv[
                    pl.ds(c * _B + h * (_B // 2) + k * 16, 16)]

    def _gather_start(c, b):
        pltpu.async_copy(
            z_hbm.at[srcv.at[pl.ds(c * _B, _B)]], rows.at[b], gsems.at[b])

    def _gather_wait(c, b):
        pltpu.make_async_copy(
            z_hbm.at[srcv.at[pl.ds(c * _B, _B)]], rows.at[b], gsems.at[b]
        ).wait()

    def _scat_start(b):
        for h in range(2):
            pltpu.async_copy(rows.at[b, pl.ds(h * (_B // 2), _B // 2)],
                             acc.at[dring.at[b, h]], ssems.at[h * _NBUF + b],
                             add=True)

    def _scat_wait(b):
        for h in range(2):
            pltpu.make_async_copy(
                rows.at[b, pl.ds(h * (_B // 2), _B // 2)],
                acc.at[dring.at[b, h]], ssems.at[h * _NBUF + b]
            ).wait()

    n_slots = ((n_full + 1) + _NBUF - 1) // _NBUF * _NBUF

    def _step(j, _):
        for u in range(_NBUF):
            c = j * _NBUF + u
            b = u

            @pl.when(jnp.logical_and(c >= _NBUF, c - _NBUF < n_full))
            def _():
                _scat_wait(b)

            @pl.when(c < n_full)
            def _():
                _gather_start(c, b)
                _stage_dst(c, b)

            bp = (u - 1) % _NBUF

            @pl.when(jnp.logical_and(c >= 1, c - 1 < n_full))
            def _():
                _gather_wait(c - 1, bp)
                _scat_start(bp)

        return 0

    lax.fori_loop(0, n_slots // _NBUF, _step, 0)

    for c in range(max(n_slots - _NBUF, 0), n_slots - 1):
        @pl.when(c < n_full)
        def _():
            _scat_wait(c % _NBUF)

    if tail:
        tbase = n_full * _B
        pltpu.async_copy(
            z_hbm.at[srcv.at[pl.ds(tbase, tail)]], rtail, gsems.at[0])
        for k in range(tail // 16):
            dtail[pl.ds(k * 16, 16)] = dstv[pl.ds(tbase + k * 16, 16)]
        pltpu.make_async_copy(
            z_hbm.at[srcv.at[pl.ds(tbase, tail)]], rtail, gsems.at[0]
        ).wait()
        pltpu.sync_copy(rtail, acc.at[dtail], add=True)

    plsc.subcore_barrier()

    def _wrun(start):
        def _go(i, _):
            ck = i * NS + sid

            @pl.when(ck < n_zfull)
            def _():
                d = pltpu.make_async_copy(
                    acc.at[pl.ds(ck * 128, 128)],
                    seg_out.at[pl.ds(cid * N + ck * 128, 128),
                               pl.ds(0, _AW)],
                    ssems.at[0])
                d.start() if start else d.wait()

            if z_tail:
                @pl.when(ck == n_zfull)
                def _():
                    d = pltpu.make_async_copy(
                        acc.at[pl.ds(n_zfull * 128, z_tail)],
                        seg_out.at[pl.ds(cid * N + n_zfull * 128, z_tail),
                                   pl.ds(0, _AW)],
                        ssems.at[0])
                    d.start() if start else d.wait()

            return 0

        lax.fori_loop(0, z_iters, _go, 0)

    _wrun(True)
    _wrun(False)


def _phase2(z, src, dst):
    N = z.shape[0]
    E = src.shape[0]
    info = plsc.get_sparse_core_info()
    NC, NS = info.num_cores, info.num_subcores
    NW = NC * NS
    assert N % 16 == 0 and E % NW == 0
    ept = E // NW
    assert ept % 8 == 0 and (ept % _B) % 16 == 0

    mesh = plsc.VectorSubcoreMesh(core_axis_name="c", subcore_axis_name="s")
    body = functools.partial(_sc_body, N, NC, NS, ept)
    tail = ept - (ept // _B) * _B
    return pl.kernel(
        body,
        out_type=jax.ShapeDtypeStruct((NC * N, 128), jnp.float32),
        mesh=mesh,
        compiler_params=pltpu.CompilerParams(use_tc_tiling_on_sc=False),
        scratch_types=(
            pltpu.VMEM((ept,), jnp.int32),
            pltpu.VMEM((ept,), jnp.int32),
            pltpu.VMEM((_NBUF, 2, _B // 2), jnp.int32),
            pltpu.VMEM((max(tail, 16),), jnp.int32),
            pltpu.VMEM((_NBUF, _B, _AW), jnp.float32),
            pltpu.VMEM((max(tail, 16), _AW), jnp.float32),
            pltpu.VMEM((128, _AW), jnp.float32),
            pltpu.VMEM_SHARED((N, _AW), jnp.float32),
            pltpu.SemaphoreType.DMA((_NBUF,)),
            pltpu.SemaphoreType.DMA((2 * _NBUF,)),
            pltpu.SemaphoreType.DMA,
        ),
    )(z, src, dst)


def _fin_body(g, CH, xw1_ref, sega_ref, segb_ref, wd_ref, bd_ref, y_ref,
              pool_ref):
    i = pl.program_id(0)
    seg = sega_ref[...] + segb_ref[...]
    cnt = seg[:, CH:CH + 1]
    out = xw1_ref[...] + seg[:, :CH] / jnp.maximum(cnt, 1.0)
    sq = jnp.sum(out * out, axis=-1, keepdims=True)
    out = out * lax.rsqrt(jnp.maximum(sq, 1e-12))
    out = jnp.maximum(out, 0.0)
    pooled = jnp.sum(out, axis=0, keepdims=True)

    @pl.when(i == 0)
    def _():
        pool_ref[...] = jnp.zeros_like(pool_ref)

    pool_ref[...] += pooled

    @pl.when(i == g - 1)
    def _():
        y_ref[...] = (
            jnp.dot(pool_ref[...], wd_ref[...],
                    preferred_element_type=jnp.float32) + bd_ref[...]
        )


def _phase3(xw1b, seg, Wd, bd2):
    CH, n_out = Wd.shape
    N = xw1b.shape[0]
    g = 5
    blk = N // g
    assert N % g == 0 and blk % 8 == 0
    return pl.pallas_call(
        functools.partial(_fin_body, g, CH),
        grid=(g,),
        in_specs=[
            pl.BlockSpec((blk, CH), lambda i: (i, 0)),
            pl.BlockSpec((blk, 128), lambda i: (i, 0)),
            pl.BlockSpec((blk, 128), lambda i: (N // blk + i, 0)),
            pl.BlockSpec((CH, n_out), lambda i: (0, 0)),
            pl.BlockSpec((1, n_out), lambda i: (0, 0)),
        ],
        out_specs=pl.BlockSpec((1, n_out), lambda i: (0, 0)),
        out_shape=jax.ShapeDtypeStruct((1, n_out), jnp.float32),
        scratch_shapes=[pltpu.VMEM((1, CH), jnp.float32)],
    )(xw1b, seg, seg, Wd, bd2)


def kernel(x, edge_index, W, b, Wd, bd):
    N, F = x.shape
    CH = W.shape[1]
    xw1b, z, src, dst = _phase1(x, W, b.reshape(1, CH), edge_index)
    seg = _phase2(z, src, dst)
    y = _phase3(xw1b, seg, Wd, bd.reshape(1, -1))
    return y.reshape(-1)


# speedup vs baseline: 1.0004x; 1.0004x over previous
import functools

import jax
import jax.numpy as jnp
from jax import lax
from jax.experimental import pallas as pl
from jax.experimental.pallas import tpu as pltpu
from jax.experimental.pallas import tpu_sc as plsc

_B = 128
_NBUF = 4
_AW = 64


def _mm_body(F, x_ref, w_ref, b_ref, ei_ref, xw1_ref, z_ref,
             src_ref, dst_ref):
    x = x_ref[...]
    w = w_ref[...]
    xw1_ref[...] = (
        jnp.dot(x, w[:F, :], preferred_element_type=jnp.float32) + b_ref[...]
    )
    z_ref[...] = jnp.dot(x, w[F:, :], preferred_element_type=jnp.float32)
    i = pl.program_id(0)
    e = ei_ref[...]
    eblk = e.shape[1]
    src_ref[pl.ds(i * eblk, eblk)] = e[0]
    dst_ref[pl.ds(i * eblk, eblk)] = e[1]


def _phase1(x, W, b2, ei):
    N, F = x.shape
    CH = W.shape[1]
    E = ei.shape[1]
    g = 5
    blk = N // g
    eblk = E // g
    assert N % g == 0 and E % g == 0 and blk % 8 == 0
    return pl.pallas_call(
        functools.partial(_mm_body, F),
        grid=(g,),
        in_specs=[
            pl.BlockSpec((blk, F), lambda i: (i, 0)),
            pl.BlockSpec((2 * F, CH), lambda i: (0, 0)),
            pl.BlockSpec((1, CH), lambda i: (0, 0)),
            pl.BlockSpec((2, eblk), lambda i: (0, i)),
        ],
        out_specs=(
            pl.BlockSpec((blk, CH), lambda i: (i, 0)),
            pl.BlockSpec((blk, _AW), lambda i: (i, 0)),
            pl.BlockSpec((E,), lambda i: (0,)),
            pl.BlockSpec((E,), lambda i: (0,)),
        ),
        out_shape=(
            jax.ShapeDtypeStruct((N, CH), jnp.float32),
            jax.ShapeDtypeStruct((N, _AW), jnp.float32),
            jax.ShapeDtypeStruct((E,), jnp.int32),
            jax.ShapeDtypeStruct((E,), jnp.int32),
        ),
    )(x, W, b2, ei)


def _sc_body(
    N, NC, NS, ept, cnt_rows,
    z_hbm, src_hbm, dst_hbm,
    seg_out, cnt_out,
    srcv, dstv, dring, dtail, rows, rtail, zbuf, cntl, ridx, cexp,
    acc, cacc, gsems, ssems, psem,
):
    cid = lax.axis_index("c")
    sid = lax.axis_index("s")
    wid = sid * NC + cid

    n_full = ept // _B
    tail = ept - n_full * _B

    zvec = jnp.zeros((16,), jnp.float32)
    iota16 = lax.iota(jnp.int32, 16)
    ones16 = jnp.ones((16,), jnp.float32)
    zidx16 = jnp.zeros((16,), jnp.int32)

    ebase = pl.multiple_of(wid * ept, 8)
    pltpu.async_copy(src_hbm.at[pl.ds(ebase, ept)], srcv, psem)
    pltpu.async_copy(dst_hbm.at[pl.ds(ebase, ept)], dstv, psem)

    for r in range(128):
        for k in range(_AW // 16):
            zbuf[r, pl.ds(k * 16, 16)] = zvec

    def _zc(r, _):
        cntl[r, :] = zvec
        return 0

    lax.fori_loop(0, cnt_rows, _zc, 0)

    for j in range(cnt_rows // _B):
        for k in range(_B // 16):
            ridx[j, pl.ds(k * 16, 16)] = iota16 + (j * _B + k * 16)

    cshare = cnt_rows // NS
    pltpu.sync_copy(
        cntl.at[pl.ds(sid * cshare, cshare)],
        cacc.at[pl.ds(sid * cshare, cshare)],
    )

    n_zfull = N // 128
    z_tail = N - n_zfull * 128
    n_zchunk = n_zfull + (1 if z_tail else 0)
    z_iters = (n_zchunk + NS - 1) // NS

    def _zrun(start):
        def _go(i, _):
            ck = i * NS + sid

            @pl.when(ck < n_zfull)
            def _():
                d = pltpu.make_async_copy(
                    zbuf, acc.at[pl.ds(ck * 128, 128)], ssems.at[0])
                d.start() if start else d.wait()

            if z_tail:
                @pl.when(ck == n_zfull)
                def _():
                    d = pltpu.make_async_copy(
                        zbuf.at[pl.ds(0, z_tail)],
                        acc.at[pl.ds(n_zfull * 128, z_tail)], ssems.at[0])
                    d.start() if start else d.wait()

            return 0

        lax.fori_loop(0, z_iters, _go, 0)

    _zrun(True)
    _zrun(False)

    pltpu.make_async_copy(src_hbm.at[pl.ds(ebase, ept)], srcv, psem).wait()
    pltpu.make_async_copy(dst_hbm.at[pl.ds(ebase, ept)], dstv, psem).wait()

    plsc.subcore_barrier()

    def _stage_dst(c, b):
        for k in range(_B // 16):
            v = dstv[pl.ds(c * _B + k * 16, 16)]
            dring[b, pl.ds(k * 16, 16)] = v
            plsc.addupdate_scatter(
                cntl,
                [lax.shift_right_logical(v, 4), lax.bitwise_and(v, 15)],
                ones16,
            )

    def _gather_start(c, b):
        pltpu.async_copy(
            z_hbm.at[srcv.at[pl.ds(c * _B, _B)]], rows.at[b], gsems.at[b])

    def _gather_wait(c, b):
        pltpu.make_async_copy(
            z_hbm.at[srcv.at[pl.ds(c * _B, _B)]], rows.at[b], gsems.at[b]
        ).wait()

    def _scat_start(b):
        pltpu.async_copy(rows.at[b], acc.at[dring.at[b]], ssems.at[b],
                         add=True)

    def _scat_wait(b):
        pltpu.make_async_copy(
            rows.at[b], acc.at[dring.at[b]], ssems.at[b]
        ).wait()

    n_slots = ((n_full + 1) + _NBUF - 1) // _NBUF * _NBUF

    def _step(j, _):
        for u in range(_NBUF):
            c = j * _NBUF + u
            b = u

            @pl.when(jnp.logical_and(c >= _NBUF, c - _NBUF < n_full))
            def _():
                _scat_wait(b)

            @pl.when(c < n_full)
            def _():
                _gather_start(c, b)
                _stage_dst(c, b)

            bp = (u - 1) % _NBUF

            @pl.when(jnp.logical_and(c >= 1, c - 1 < n_full))
            def _():
                _gather_wait(c - 1, bp)
                _scat_start(bp)

        return 0

    lax.fori_loop(0, n_slots // _NBUF, _step, 0)

    for c in range(max(n_slots - _NBUF, 0), n_slots - 1):
        @pl.when(c < n_full)
        def _():
            _scat_wait(c % _NBUF)

    if tail:
        tbase = n_full * _B
        pltpu.async_copy(
            z_hbm.at[srcv.at[pl.ds(tbase, tail)]], rtail, gsems.at[0])
        for k in range(tail // 16):
            v = dstv[pl.ds(tbase + k * 16, 16)]
            dtail[pl.ds(k * 16, 16)] = v
            plsc.addupdate_scatter(
                cntl,
                [lax.shift_right_logical(v, 4), lax.bitwise_and(v, 15)],
                ones16,
            )
        pltpu.make_async_copy(
            z_hbm.at[srcv.at[pl.ds(tbase, tail)]], rtail, gsems.at[0]
        ).wait()
        pltpu.sync_copy(rtail, acc.at[dtail], add=True)

    for j in range(cnt_rows // _B):
        pltpu.sync_copy(
            cntl.at[pl.ds(j * _B, _B)], cacc.at[ridx.at[j]], add=True)

    plsc.subcore_barrier()

    def _wrun(start):
        def _go(i, _):
            ck = i * NS + sid

            @pl.when(ck < n_zfull)
            def _():
                d = pltpu.make_async_copy(
                    acc.at[pl.ds(ck * 128, 128)],
                    seg_out.at[pl.ds(ck * 128, 128),
                               pl.ds(cid * _AW, _AW)],
                    ssems.at[0])
                d.start() if start else d.wait()

            if z_tail:
                @pl.when(ck == n_zfull)
                def _():
                    d = pltpu.make_async_copy(
                        acc.at[pl.ds(n_zfull * 128, z_tail)],
                        seg_out.at[pl.ds(n_zfull * 128, z_tail),
                                   pl.ds(cid * _AW, _AW)],
                        ssems.at[0])
                    d.start() if start else d.wait()

            return 0

        lax.fori_loop(0, z_iters, _go, 0)

    _wrun(True)
    _wrun(False)

    pltpu.sync_copy(cacc.at[pl.ds(sid * cshare, cshare)],
                    cntl.at[pl.ds(0, cshare)])
    for k in range(cshare):
        c16 = cntl[k, :]
        plsc.store_scatter(cexp, [iota16 + k * 16, zidx16], c16)
    pltpu.sync_copy(
        cexp,
        cnt_out.at[pl.ds((cid * NS + sid) * cshare * 16, cshare * 16)])


def _phase2(z, src, dst):
    N = z.shape[0]
    E = src.shape[0]
    info = plsc.get_sparse_core_info()
    NC, NS = info.num_cores, info.num_subcores
    NW = NC * NS
    assert N % 16 == 0 and E % NW == 0
    ept = E // NW
    assert ept % 8 == 0 and (ept % _B) % 16 == 0
    cnt_rows = (N // 16 + _B - 1) // _B * _B
    assert cnt_rows % NS == 0

    mesh = plsc.VectorSubcoreMesh(core_axis_name="c", subcore_axis_name="s")
    body = functools.partial(_sc_body, N, NC, NS, ept, cnt_rows)
    tail = ept - (ept // _B) * _B
    cshare = cnt_rows // NS
    return pl.kernel(
        body,
        out_type=(
            jax.ShapeDtypeStruct((N, 128), jnp.float32),
            jax.ShapeDtypeStruct((NC * cnt_rows * 16, 16), jnp.float32),
        ),
        mesh=mesh,
        compiler_params=pltpu.CompilerParams(use_tc_tiling_on_sc=False,
                                             needs_layout_passes=False),
        scratch_types=(
            pltpu.VMEM((ept,), jnp.int32),
            pltpu.VMEM((ept,), jnp.int32),
            pltpu.VMEM((_NBUF, _B), jnp.int32),
            pltpu.VMEM((max(tail, 16),), jnp.int32),
            pltpu.VMEM((_NBUF, _B, _AW), jnp.float32),
            pltpu.VMEM((max(tail, 16), _AW), jnp.float32),
            pltpu.VMEM((128, _AW), jnp.float32),
            pltpu.VMEM((cnt_rows, 16), jnp.float32),
            pltpu.VMEM((cnt_rows // _B, _B), jnp.int32),
            pltpu.VMEM((cshare * 16, 16), jnp.float32),
            pltpu.VMEM_SHARED((N, _AW), jnp.float32),
            pltpu.VMEM_SHARED((cnt_rows, 16), jnp.float32),
            pltpu.SemaphoreType.DMA((_NBUF,)),
            pltpu.SemaphoreType.DMA((_NBUF,)),
            pltpu.SemaphoreType.DMA,
        ),
    )(z, src, dst)


def _fin_body(g, CH, xw1_ref, seg_ref, cnta_ref, cntb_ref, wd_ref, bd_ref,
              y_ref, pool_ref):
    i = pl.program_id(0)
    seg = seg_ref[...]
    seg = seg[:, :CH] + seg[:, CH:2 * CH]
    cnt = cnta_ref[0][:, :1] + cntb_ref[0][:, :1]
    out = xw1_ref[...] + seg / jnp.maximum(cnt, 1.0)
    sq = jnp.sum(out * out, axis=-1, keepdims=True)
    out = out * lax.rsqrt(jnp.maximum(sq, 1e-12))
    out = jnp.maximum(out, 0.0)
    pooled = jnp.sum(out, axis=0, keepdims=True)

    @pl.when(i == 0)
    def _():
        pool_ref[...] = jnp.zeros_like(pool_ref)

    pool_ref[...] += pooled

    @pl.when(i == g - 1)
    def _():
        y_ref[...] = (
            jnp.dot(pool_ref[...], wd_ref[...],
                    preferred_element_type=jnp.float32) + bd_ref[...]
        )


def _phase3(xw1b, seg, cnt3, Wd, bd2):
    CH, n_out = Wd.shape
    N = xw1b.shape[0]
    g = 5
    blk = N // g
    assert N % g == 0 and blk % 8 == 0
    return pl.pallas_call(
        functools.partial(_fin_body, g, CH),
        grid=(g,),
        in_specs=[
            pl.BlockSpec((blk, CH), lambda i: (i, 0)),
            pl.BlockSpec((blk, 128), lambda i: (i, 0)),
            pl.BlockSpec((1, blk, 16), lambda i: (0, i, 0)),
            pl.BlockSpec((1, blk, 16), lambda i: (1, i, 0)),
            pl.BlockSpec((CH, n_out), lambda i: (0, 0)),
            pl.BlockSpec((1, n_out), lambda i: (0, 0)),
        ],
        out_specs=pl.BlockSpec((1, n_out), lambda i: (0, 0)),
        out_shape=jax.ShapeDtypeStruct((1, n_out), jnp.float32),
        scratch_shapes=[pltpu.VMEM((1, CH), jnp.float32)],
    )(xw1b, seg, cnt3, cnt3, Wd, bd2)


def kernel(x, edge_index, W, b, Wd, bd):
    N, F = x.shape
    CH = W.shape[1]
    xw1b, z, src, dst = _phase1(x, W, b.reshape(1, CH), edge_index)
    seg, cnt = _phase2(z, src, dst)
    npad = cnt.shape[0] // 2
    cnt3 = cnt.reshape(2, npad, 16)
    y = _phase3(xw1b, seg, cnt3, Wd, bd.reshape(1, -1))
    return y.reshape(-1)


# speedup vs baseline: 1.0695x; 1.0691x over previous
import functools

import jax
import jax.numpy as jnp
from jax import lax
from jax.experimental import pallas as pl
from jax.experimental.pallas import tpu as pltpu
from jax.experimental.pallas import tpu_sc as plsc

_B = 128
_NBUF = 4
_AW = 64


def _mm_body(F, x_ref, w_ref, b_ref, ei_ref, xw1_ref, z_ref,
             src_ref, dst_ref):
    x = x_ref[...]
    w = w_ref[...]
    xw1_ref[...] = (
        jnp.dot(x, w[:F, :], preferred_element_type=jnp.float32) + b_ref[...]
    )
    z_ref[...] = jnp.dot(x, w[F:, :], preferred_element_type=jnp.float32)
    i = pl.program_id(0)
    e = ei_ref[...]
    eblk = e.shape[1]
    src_ref[pl.ds(i * eblk, eblk)] = e[0]
    dst_ref[pl.ds(i * eblk, eblk)] = e[1]


def _phase1(x, W, b2, ei):
    N, F = x.shape
    CH = W.shape[1]
    E = ei.shape[1]
    g = 5
    blk = N // g
    eblk = E // g
    assert N % g == 0 and E % g == 0 and blk % 8 == 0
    return pl.pallas_call(
        functools.partial(_mm_body, F),
        grid=(g,),
        in_specs=[
            pl.BlockSpec((blk, F), lambda i: (i, 0)),
            pl.BlockSpec((2 * F, CH), lambda i: (0, 0)),
            pl.BlockSpec((1, CH), lambda i: (0, 0)),
            pl.BlockSpec((2, eblk), lambda i: (0, i)),
        ],
        out_specs=(
            pl.BlockSpec((blk, CH), lambda i: (i, 0)),
            pl.BlockSpec((blk, _AW), lambda i: (i, 0)),
            pl.BlockSpec((E,), lambda i: (0,)),
            pl.BlockSpec((E,), lambda i: (0,)),
        ),
        out_shape=(
            jax.ShapeDtypeStruct((N, CH), jnp.float32),
            jax.ShapeDtypeStruct((N, _AW), jnp.float32),
            jax.ShapeDtypeStruct((E,), jnp.int32),
            jax.ShapeDtypeStruct((E,), jnp.int32),
        ),
    )(x, W, b2, ei)


def _sc_body(
    N, NC, NS, ept, cnt_rows,
    z_hbm, src_hbm, dst_hbm,
    seg_out, cnt_out,
    srcv, dstv, dring, dtail, rows, rtail, zbuf, cntl, ridx, cexp,
    acc, cacc, gsems, ssems, psem,
):
    cid = lax.axis_index("c")
    sid = lax.axis_index("s")
    wid = sid * NC + cid

    n_full = ept // _B
    tail = ept - n_full * _B

    zvec = jnp.zeros((16,), jnp.float32)
    iota16 = lax.iota(jnp.int32, 16)
    ones16 = jnp.ones((16,), jnp.float32)
    zidx16 = jnp.zeros((16,), jnp.int32)

    ebase = pl.multiple_of(wid * ept, 8)
    pltpu.async_copy(src_hbm.at[pl.ds(ebase, ept)], srcv, psem)
    pltpu.async_copy(dst_hbm.at[pl.ds(ebase, ept)], dstv, psem)

    for r in range(128):
        for k in range(_AW // 16):
            zbuf[r, pl.ds(k * 16, 16)] = zvec

    def _zc(r, _):
        cntl[r, :] = zvec
        return 0

    lax.fori_loop(0, cnt_rows, _zc, 0)

    for j in range(cnt_rows // _B):
        for k in range(_B // 16):
            ridx[j, pl.ds(k * 16, 16)] = iota16 + (j * _B + k * 16)

    cshare = cnt_rows // NS
    pltpu.sync_copy(
        cntl.at[pl.ds(sid * cshare, cshare)],
        cacc.at[pl.ds(sid * cshare, cshare)],
    )

    n_zfull = N // 128
    z_tail = N - n_zfull * 128
    n_zchunk = n_zfull + (1 if z_tail else 0)
    z_iters = (n_zchunk + NS - 1) // NS

    def _zrun(start):
        def _go(i, _):
            ck = i * NS + sid

            @pl.when(ck < n_zfull)
            def _():
                d = pltpu.make_async_copy(
                    zbuf, acc.at[pl.ds(ck * 128, 128)], ssems.at[0])
                d.start() if start else d.wait()

            if z_tail:
                @pl.when(ck == n_zfull)
                def _():
                    d = pltpu.make_async_copy(
                        zbuf.at[pl.ds(0, z_tail)],
                        acc.at[pl.ds(n_zfull * 128, z_tail)], ssems.at[0])
                    d.start() if start else d.wait()

            return 0

        lax.fori_loop(0, z_iters, _go, 0)

    _zrun(True)
    _zrun(False)

    pltpu.make_async_copy(src_hbm.at[pl.ds(ebase, ept)], srcv, psem).wait()
    pltpu.make_async_copy(dst_hbm.at[pl.ds(ebase, ept)], dstv, psem).wait()

    plsc.subcore_barrier()

    def _stage_dst(c, b):
        for k in range(_B // 16):
            v = dstv[pl.ds(c * _B + k * 16, 16)]
            dring[b, pl.ds(k * 16, 16)] = v
            plsc.addupdate_scatter(
                cntl,
                [lax.shift_right_logical(v, 4), lax.bitwise_and(v, 15)],
                ones16,
            )

    def _gather_start(c, b):
        pltpu.async_copy(
            z_hbm.at[srcv.at[pl.ds(c * _B, _B)]], rows.at[b], gsems.at[b])

    def _gather_wait(c, b):
        pltpu.make_async_copy(
            z_hbm.at[srcv.at[pl.ds(c * _B, _B)]], rows.at[b], gsems.at[b]
        ).wait()

    def _scat_start(b):
        pltpu.async_copy(rows.at[b], acc.at[dring.at[b]], ssems.at[b],
                         add=True)

    def _scat_wait(b):
        pltpu.make_async_copy(
            rows.at[b], acc.at[dring.at[b]], ssems.at[b]
        ).wait()

    n_slots = ((n_full + 2) + _NBUF - 1) // _NBUF * _NBUF

    def _step(j, _):
        for u in range(_NBUF):
            c = j * _NBUF + u
            b = u

            @pl.when(jnp.logical_and(c >= _NBUF, c - _NBUF < n_full))
            def _():
                _scat_wait(b)

            @pl.when(c < n_full)
            def _():
                _gather_start(c, b)
                _stage_dst(c, b)

            bp = (u - 2) % _NBUF

            @pl.when(jnp.logical_and(c >= 2, c - 2 < n_full))
            def _():
                _gather_wait(c - 2, bp)
                _scat_start(bp)

        return 0

    lax.fori_loop(0, n_slots // _NBUF, _step, 0)

    for c in range(max(n_slots - _NBUF, 0), n_slots - 1):
        @pl.when(c < n_full)
        def _():
            _scat_wait(c % _NBUF)

    if tail:
        tbase = n_full * _B
        pltpu.async_copy(
            z_hbm.at[srcv.at[pl.ds(tbase, tail)]], rtail, gsems.at[0])
        for k in range(tail // 16):
            v = dstv[pl.ds(tbase + k * 16, 16)]
            dtail[pl.ds(k * 16, 16)] = v
            plsc.addupdate_scatter(
                cntl,
                [lax.shift_right_logical(v, 4), lax.bitwise_and(v, 15)],
                ones16,
            )
        pltpu.make_async_copy(
            z_hbm.at[srcv.at[pl.ds(tbase, tail)]], rtail, gsems.at[0]
        ).wait()
        pltpu.sync_copy(rtail, acc.at[dtail], add=True)

    for j in range(cnt_rows // _B):
        pltpu.sync_copy(
            cntl.at[pl.ds(j * _B, _B)], cacc.at[ridx.at[j]], add=True)

    plsc.subcore_barrier()

    def _wrun(start):
        def _go(i, _):
            ck = i * NS + sid

            @pl.when(ck < n_zfull)
            def _():
                d = pltpu.make_async_copy(
                    acc.at[pl.ds(ck * 128, 128)],
                    seg_out.at[pl.ds(ck * 128, 128),
                               pl.ds(cid * _AW, _AW)],
                    ssems.at[0])
                d.start() if start else d.wait()

            if z_tail:
                @pl.when(ck == n_zfull)
                def _():
                    d = pltpu.make_async_copy(
                        acc.at[pl.ds(n_zfull * 128, z_tail)],
                        seg_out.at[pl.ds(n_zfull * 128, z_tail),
                                   pl.ds(cid * _AW, _AW)],
                        ssems.at[0])
                    d.start() if start else d.wait()

            return 0

        lax.fori_loop(0, z_iters, _go, 0)

    _wrun(True)
    _wrun(False)

    pltpu.sync_copy(cacc.at[pl.ds(sid * cshare, cshare)],
                    cntl.at[pl.ds(0, cshare)])
    for k in range(cshare):
        c16 = cntl[k, :]
        plsc.store_scatter(cexp, [iota16 + k * 16, zidx16], c16)
    pltpu.sync_copy(
        cexp,
        cnt_out.at[pl.ds((cid * NS + sid) * cshare * 16, cshare * 16)])


def _phase2(z, src, dst):
    N = z.shape[0]
    E = src.shape[0]
    info = plsc.get_sparse_core_info()
    NC, NS = info.num_cores, info.num_subcores
    NW = NC * NS
    assert N % 16 == 0 and E % NW == 0
    ept = E // NW
    assert ept % 8 == 0 and (ept % _B) % 16 == 0
    cnt_rows = (N // 16 + _B - 1) // _B * _B
    assert cnt_rows % NS == 0

    mesh = plsc.VectorSubcoreMesh(core_axis_name="c", subcore_axis_name="s")
    body = functools.partial(_sc_body, N, NC, NS, ept, cnt_rows)
    tail = ept - (ept // _B) * _B
    cshare = cnt_rows // NS
    return pl.kernel(
        body,
        out_type=(
            jax.ShapeDtypeStruct((N, 128), jnp.float32),
            jax.ShapeDtypeStruct((NC * cnt_rows * 16, 16), jnp.float32),
        ),
        mesh=mesh,
        compiler_params=pltpu.CompilerParams(use_tc_tiling_on_sc=False,
                                             needs_layout_passes=False),
        scratch_types=(
            pltpu.VMEM((ept,), jnp.int32),
            pltpu.VMEM((ept,), jnp.int32),
            pltpu.VMEM((_NBUF, _B), jnp.int32),
            pltpu.VMEM((max(tail, 16),), jnp.int32),
            pltpu.VMEM((_NBUF, _B, _AW), jnp.float32),
            pltpu.VMEM((max(tail, 16), _AW), jnp.float32),
            pltpu.VMEM((128, _AW), jnp.float32),
            pltpu.VMEM((cnt_rows, 16), jnp.float32),
            pltpu.VMEM((cnt_rows // _B, _B), jnp.int32),
            pltpu.VMEM((cshare * 16, 16), jnp.float32),
            pltpu.VMEM_SHARED((N, _AW), jnp.float32),
            pltpu.VMEM_SHARED((cnt_rows, 16), jnp.float32),
            pltpu.SemaphoreType.DMA((_NBUF,)),
            pltpu.SemaphoreType.DMA((_NBUF,)),
            pltpu.SemaphoreType.DMA,
        ),
    )(z, src, dst)


def _fin_body(g, CH, xw1_ref, seg_ref, cnta_ref, cntb_ref, wd_ref, bd_ref,
              y_ref, pool_ref):
    i = pl.program_id(0)
    seg = seg_ref[...]
    seg = seg[:, :CH] + seg[:, CH:2 * CH]
    cnt = cnta_ref[0][:, :1] + cntb_ref[0][:, :1]
    out = xw1_ref[...] + seg / jnp.maximum(cnt, 1.0)
    sq = jnp.sum(out * out, axis=-1, keepdims=True)
    out = out * lax.rsqrt(jnp.maximum(sq, 1e-12))
    out = jnp.maximum(out, 0.0)
    pooled = jnp.sum(out, axis=0, keepdims=True)

    @pl.when(i == 0)
    def _():
        pool_ref[...] = jnp.zeros_like(pool_ref)

    pool_ref[...] += pooled

    @pl.when(i == g - 1)
    def _():
        y_ref[...] = (
            jnp.dot(pool_ref[...], wd_ref[...],
                    preferred_element_type=jnp.float32) + bd_ref[...]
        )


def _phase3(xw1b, seg, cnt3, Wd, bd2):
    CH, n_out = Wd.shape
    N = xw1b.shape[0]
    g = 5
    blk = N // g
    assert N % g == 0 and blk % 8 == 0
    return pl.pallas_call(
        functools.partial(_fin_body, g, CH),
        grid=(g,),
        in_specs=[
            pl.BlockSpec((blk, CH), lambda i: (i, 0)),
            pl.BlockSpec((blk, 128), lambda i: (i, 0)),
            pl.BlockSpec((1, blk, 16), lambda i: (0, i, 0)),
            pl.BlockSpec((1, blk, 16), lambda i: (1, i, 0)),
            pl.BlockSpec((CH, n_out), lambda i: (0, 0)),
            pl.BlockSpec((1, n_out), lambda i: (0, 0)),
        ],
        out_specs=pl.BlockSpec((1, n_out), lambda i: (0, 0)),
        out_shape=jax.ShapeDtypeStruct((1, n_out), jnp.float32),
        scratch_shapes=[pltpu.VMEM((1, CH), jnp.float32)],
    )(xw1b, seg, cnt3, cnt3, Wd, bd2)


def kernel(x, edge_index, W, b, Wd, bd):
    N, F = x.shape
    CH = W.shape[1]
    xw1b, z, src, dst = _phase1(x, W, b.reshape(1, CH), edge_index)
    seg, cnt = _phase2(z, src, dst)
    npad = cnt.shape[0] // 2
    cnt3 = cnt.reshape(2, npad, 16)
    y = _phase3(xw1b, seg, cnt3, Wd, bd.reshape(1, -1))
    return y.reshape(-1)


# speedup vs baseline: 1.1082x; 1.0362x over previous
import functools

import jax
import jax.numpy as jnp
from jax import lax
from jax.experimental import pallas as pl
from jax.experimental.pallas import tpu as pltpu
from jax.experimental.pallas import tpu_sc as plsc

_B = 128
_NBUF = 5
_LEAD = 3
_AW = 64


def _mm_body(F, x_ref, w_ref, b_ref, ei_ref, xw1_ref, z_ref,
             src_ref, dst_ref):
    x = x_ref[...]
    w = w_ref[...]
    xw1_ref[...] = (
        jnp.dot(x, w[:F, :], preferred_element_type=jnp.float32) + b_ref[...]
    )
    z_ref[...] = jnp.dot(x, w[F:, :], preferred_element_type=jnp.float32)
    i = pl.program_id(0)
    e = ei_ref[...]
    eblk = e.shape[1]
    src_ref[pl.ds(i * eblk, eblk)] = e[0]
    dst_ref[pl.ds(i * eblk, eblk)] = e[1]


def _phase1(x, W, b2, ei):
    N, F = x.shape
    CH = W.shape[1]
    E = ei.shape[1]
    g = 5
    blk = N // g
    eblk = E // g
    assert N % g == 0 and E % g == 0 and blk % 8 == 0
    return pl.pallas_call(
        functools.partial(_mm_body, F),
        grid=(g,),
        in_specs=[
            pl.BlockSpec((blk, F), lambda i: (i, 0)),
            pl.BlockSpec((2 * F, CH), lambda i: (0, 0)),
            pl.BlockSpec((1, CH), lambda i: (0, 0)),
            pl.BlockSpec((2, eblk), lambda i: (0, i)),
        ],
        out_specs=(
            pl.BlockSpec((blk, CH), lambda i: (i, 0)),
            pl.BlockSpec((blk, _AW), lambda i: (i, 0)),
            pl.BlockSpec((E,), lambda i: (0,)),
            pl.BlockSpec((E,), lambda i: (0,)),
        ),
        out_shape=(
            jax.ShapeDtypeStruct((N, CH), jnp.float32),
            jax.ShapeDtypeStruct((N, _AW), jnp.float32),
            jax.ShapeDtypeStruct((E,), jnp.int32),
            jax.ShapeDtypeStruct((E,), jnp.int32),
        ),
    )(x, W, b2, ei)


def _sc_body(
    N, NC, NS, ept, cnt_rows,
    z_hbm, src_hbm, dst_hbm,
    seg_out, cnt_out,
    srcv, dstv, dring, dtail, rows, rtail, zbuf, cntl, ridx, cexp,
    acc, cacc, gsems, ssems, psem,
):
    cid = lax.axis_index("c")
    sid = lax.axis_index("s")
    wid = sid * NC + cid

    n_full = ept // _B
    tail = ept - n_full * _B

    zvec = jnp.zeros((16,), jnp.float32)
    iota16 = lax.iota(jnp.int32, 16)
    ones16 = jnp.ones((16,), jnp.float32)
    zidx16 = jnp.zeros((16,), jnp.int32)

    ebase = pl.multiple_of(wid * ept, 8)
    pltpu.async_copy(src_hbm.at[pl.ds(ebase, ept)], srcv, psem)
    pltpu.async_copy(dst_hbm.at[pl.ds(ebase, ept)], dstv, psem)

    for r in range(64):
        for k in range(_AW // 16):
            zbuf[r, pl.ds(k * 16, 16)] = zvec

    def _zc(r, _):
        cntl[r, :] = zvec
        return 0

    lax.fori_loop(0, cnt_rows, _zc, 0)

    for j in range(cnt_rows // _B):
        for k in range(_B // 16):
            ridx[j, pl.ds(k * 16, 16)] = iota16 + (j * _B + k * 16)

    cshare = cnt_rows // NS
    pltpu.sync_copy(
        cntl.at[pl.ds(sid * cshare, cshare)],
        cacc.at[pl.ds(sid * cshare, cshare)],
    )

    n_zfull = N // 64
    z_tail = N - n_zfull * 64
    n_zchunk = n_zfull + (1 if z_tail else 0)
    z_iters = (n_zchunk + NS - 1) // NS

    def _zrun(start):
        def _go(i, _):
            ck = i * NS + sid

            @pl.when(ck < n_zfull)
            def _():
                d = pltpu.make_async_copy(
                    zbuf, acc.at[pl.ds(ck * 64, 64)], ssems.at[0])
                d.start() if start else d.wait()

            if z_tail:
                @pl.when(ck == n_zfull)
                def _():
                    d = pltpu.make_async_copy(
                        zbuf.at[pl.ds(0, z_tail)],
                        acc.at[pl.ds(n_zfull * 64, z_tail)], ssems.at[0])
                    d.start() if start else d.wait()

            return 0

        lax.fori_loop(0, z_iters, _go, 0)

    _zrun(True)
    _zrun(False)

    pltpu.make_async_copy(src_hbm.at[pl.ds(ebase, ept)], srcv, psem).wait()
    pltpu.make_async_copy(dst_hbm.at[pl.ds(ebase, ept)], dstv, psem).wait()

    plsc.subcore_barrier()

    def _stage_dst(c, b):
        for k in range(_B // 16):
            v = dstv[pl.ds(c * _B + k * 16, 16)]
            dring[b, pl.ds(k * 16, 16)] = v
            plsc.addupdate_scatter(
                cntl,
                [lax.shift_right_logical(v, 4), lax.bitwise_and(v, 15)],
                ones16,
            )

    def _gather_start(c, b):
        pltpu.async_copy(
            z_hbm.at[srcv.at[pl.ds(c * _B, _B)]], rows.at[b], gsems.at[b])

    def _gather_wait(c, b):
        pltpu.make_async_copy(
            z_hbm.at[srcv.at[pl.ds(c * _B, _B)]], rows.at[b], gsems.at[b]
        ).wait()

    def _scat_start(b):
        pltpu.async_copy(rows.at[b], acc.at[dring.at[b]], ssems.at[b],
                         add=True)

    def _scat_wait(b):
        pltpu.make_async_copy(
            rows.at[b], acc.at[dring.at[b]], ssems.at[b]
        ).wait()

    n_slots = ((n_full + _LEAD) + _NBUF - 1) // _NBUF * _NBUF

    def _step(j, _):
        for u in range(_NBUF):
            c = j * _NBUF + u
            b = u

            @pl.when(jnp.logical_and(c >= _NBUF, c - _NBUF < n_full))
            def _():
                _scat_wait(b)

            @pl.when(c < n_full)
            def _():
                _gather_start(c, b)
                _stage_dst(c, b)

            bp = (u - _LEAD) % _NBUF

            @pl.when(jnp.logical_and(c >= _LEAD, c - _LEAD < n_full))
            def _():
                _gather_wait(c - _LEAD, bp)
                _scat_start(bp)

        return 0

    lax.fori_loop(0, n_slots // _NBUF, _step, 0)

    for c in range(max(n_slots - _NBUF, 0), n_slots - 1):
        @pl.when(c < n_full)
        def _():
            _scat_wait(c % _NBUF)

    if tail:
        tbase = n_full * _B
        pltpu.async_copy(
            z_hbm.at[srcv.at[pl.ds(tbase, tail)]], rtail, gsems.at[0])
        for k in range(tail // 16):
            v = dstv[pl.ds(tbase + k * 16, 16)]
            dtail[pl.ds(k * 16, 16)] = v
            plsc.addupdate_scatter(
                cntl,
                [lax.shift_right_logical(v, 4), lax.bitwise_and(v, 15)],
                ones16,
            )
        pltpu.make_async_copy(
            z_hbm.at[srcv.at[pl.ds(tbase, tail)]], rtail, gsems.at[0]
        ).wait()
        pltpu.sync_copy(rtail, acc.at[dtail], add=True)

    for j in range(cnt_rows // _B):
        pltpu.sync_copy(
            cntl.at[pl.ds(j * _B, _B)], cacc.at[ridx.at[j]], add=True)

    plsc.subcore_barrier()

    def _wrun(start):
        def _go(i, _):
            ck = i * NS + sid

            @pl.when(ck < n_zfull)
            def _():
                d = pltpu.make_async_copy(
                    acc.at[pl.ds(ck * 64, 64)],
                    seg_out.at[pl.ds(ck * 64, 64),
                               pl.ds(cid * _AW, _AW)],
                    ssems.at[0])
                d.start() if start else d.wait()

            if z_tail:
                @pl.when(ck == n_zfull)
                def _():
                    d = pltpu.make_async_copy(
                        acc.at[pl.ds(n_zfull * 64, z_tail)],
                        seg_out.at[pl.ds(n_zfull * 64, z_tail),
                                   pl.ds(cid * _AW, _AW)],
                        ssems.at[0])
                    d.start() if start else d.wait()

            return 0

        lax.fori_loop(0, z_iters, _go, 0)

    _wrun(True)
    _wrun(False)

    pltpu.sync_copy(cacc.at[pl.ds(sid * cshare, cshare)],
                    cntl.at[pl.ds(0, cshare)])
    for k in range(cshare):
        c16 = cntl[k, :]
        plsc.store_scatter(cexp, [iota16 + k * 16, zidx16], c16)
    pltpu.sync_copy(
        cexp,
        cnt_out.at[pl.ds((cid * NS + sid) * cshare * 16, cshare * 16)])


def _phase2(z, src, dst):
    N = z.shape[0]
    E = src.shape[0]
    info = plsc.get_sparse_core_info()
    NC, NS = info.num_cores, info.num_subcores
    NW = NC * NS
    assert N % 16 == 0 and E % NW == 0
    ept = E // NW
    assert ept % 8 == 0 and (ept % _B) % 16 == 0
    cnt_rows = (N // 16 + _B - 1) // _B * _B
    assert cnt_rows % NS == 0

    mesh = plsc.VectorSubcoreMesh(core_axis_name="c", subcore_axis_name="s")
    body = functools.partial(_sc_body, N, NC, NS, ept, cnt_rows)
    tail = ept - (ept // _B) * _B
    cshare = cnt_rows // NS
    return pl.kernel(
        body,
        out_type=(
            jax.ShapeDtypeStruct((N, 128), jnp.float32),
            jax.ShapeDtypeStruct((NC * cnt_rows * 16, 16), jnp.float32),
        ),
        mesh=mesh,
        compiler_params=pltpu.CompilerParams(use_tc_tiling_on_sc=False,
                                             needs_layout_passes=False),
        scratch_types=(
            pltpu.VMEM((ept,), jnp.int32),
            pltpu.VMEM((ept,), jnp.int32),
            pltpu.VMEM((_NBUF, _B), jnp.int32),
            pltpu.VMEM((max(tail, 16),), jnp.int32),
            pltpu.VMEM((_NBUF, _B, _AW), jnp.float32),
            pltpu.VMEM((max(tail, 16), _AW), jnp.float32),
            pltpu.VMEM((64, _AW), jnp.float32),
            pltpu.VMEM((cnt_rows, 16), jnp.float32),
            pltpu.VMEM((cnt_rows // _B, _B), jnp.int32),
            pltpu.VMEM((cshare * 16, 16), jnp.float32),
            pltpu.VMEM_SHARED((N, _AW), jnp.float32),
            pltpu.VMEM_SHARED((cnt_rows, 16), jnp.float32),
            pltpu.SemaphoreType.DMA((_NBUF,)),
            pltpu.SemaphoreType.DMA((_NBUF,)),
            pltpu.SemaphoreType.DMA,
        ),
    )(z, src, dst)


def _fin_body(g, CH, xw1_ref, seg_ref, cnta_ref, cntb_ref, wd_ref, bd_ref,
              y_ref, pool_ref):
    i = pl.program_id(0)
    seg = seg_ref[...]
    seg = seg[:, :CH] + seg[:, CH:2 * CH]
    cnt = cnta_ref[0][:, :1] + cntb_ref[0][:, :1]
    out = xw1_ref[...] + seg / jnp.maximum(cnt, 1.0)
    sq = jnp.sum(out * out, axis=-1, keepdims=True)
    out = out * lax.rsqrt(jnp.maximum(sq, 1e-12))
    out = jnp.maximum(out, 0.0)
    pooled = jnp.sum(out, axis=0, keepdims=True)

    @pl.when(i == 0)
    def _():
        pool_ref[...] = jnp.zeros_like(pool_ref)

    pool_ref[...] += pooled

    @pl.when(i == g - 1)
    def _():
        y_ref[...] = (
            jnp.dot(pool_ref[...], wd_ref[...],
                    preferred_element_type=jnp.float32) + bd_ref[...]
        )


def _phase3(xw1b, seg, cnt3, Wd, bd2):
    CH, n_out = Wd.shape
    N = xw1b.shape[0]
    g = 5
    blk = N // g
    assert N % g == 0 and blk % 8 == 0
    return pl.pallas_call(
        functools.partial(_fin_body, g, CH),
        grid=(g,),
        in_specs=[
            pl.BlockSpec((blk, CH), lambda i: (i, 0)),
            pl.BlockSpec((blk, 128), lambda i: (i, 0)),
            pl.BlockSpec((1, blk, 16), lambda i: (0, i, 0)),
            pl.BlockSpec((1, blk, 16), lambda i: (1, i, 0)),
            pl.BlockSpec((CH, n_out), lambda i: (0, 0)),
            pl.BlockSpec((1, n_out), lambda i: (0, 0)),
        ],
        out_specs=pl.BlockSpec((1, n_out), lambda i: (0, 0)),
        out_shape=jax.ShapeDtypeStruct((1, n_out), jnp.float32),
        scratch_shapes=[pltpu.VMEM((1, CH), jnp.float32)],
    )(xw1b, seg, cnt3, cnt3, Wd, bd2)


def kernel(x, edge_index, W, b, Wd, bd):
    N, F = x.shape
    CH = W.shape[1]
    xw1b, z, src, dst = _phase1(x, W, b.reshape(1, CH), edge_index)
    seg, cnt = _phase2(z, src, dst)
    npad = cnt.shape[0] // 2
    cnt3 = cnt.reshape(2, npad, 16)
    y = _phase3(xw1b, seg, cnt3, Wd, bd.reshape(1, -1))
    return y.reshape(-1)
